# baseline (speedup 1.0000x reference)
import functools

import jax
import jax.numpy as jnp
from jax import lax
from jax.experimental import pallas as pl
from jax.experimental.pallas import tpu as pltpu
from jax.experimental.pallas import tpu_sc as plsc

B, N, M = 8, 4096, 1024
C1, C2 = 128, 256
CO = 128
BLK = 1024
NB = N // BLK
ABLK = 1024
ANB = N // ABLK


def _three_nn_body(u_ref, kn_ref, kf_ref, w_ref, fi_ref, wt_ref, pt_ref):
    b = pl.program_id(0)
    nb = pl.program_id(1)

    @pl.when(nb == 0)
    def _():
        pt_ref[0] = lax.dot_general(
            kf_ref[0], w_ref[:, :C2], (((0,), (1,)), ((), ())),
            preferred_element_type=jnp.float32,
        )

    U = u_ref[0]
    K = kn_ref[0]
    kn2 = jnp.sum(K * K, axis=1, keepdims=True)
    G = lax.dot_general(
        K, U, (((1,), (1,)), ((), ())),
        preferred_element_type=jnp.float32,
        precision=lax.Precision.HIGHEST,
    )
    d2 = kn2 - 2.0 * G
    u2 = lax.dot_general(
        jnp.ones((1, 3), jnp.float32), U * U, (((1,), (1,)), ((), ())),
        preferred_element_type=jnp.float32,
        precision=lax.Precision.HIGHEST,
    )
    iota = lax.broadcasted_iota(jnp.int32, (M, ABLK), 0)
    masks, recips = [], []
    for _ in range(3):
        dmin = jnp.min(d2, axis=0, keepdims=True)
        eq = d2 == dmin
        d2 = jnp.where(eq, 1e30, d2)
        dist = jnp.sqrt(jnp.maximum(dmin + u2, 1e-12))
        masks.append(eq)
        recips.append(1.0 / (dist + 1e-8))
    packed = (
        jnp.where(masks[0], iota, 0)
        + jnp.where(masks[1], iota * 1024, 0)
        + jnp.where(masks[2], iota * (1024 * 1024), 0)
    )
    psum = jnp.sum(packed, axis=0, keepdims=True)
    idxs = [psum & 1023, (psum >> 10) & 1023, (psum >> 20) & 1023]
    norm = recips[0] + recips[1] + recips[2]
    flat = jnp.clip(jnp.concatenate(idxs, axis=0), 0, M - 1) + b * M
    wts = jnp.concatenate(recips, axis=0) / norm
    fi_ref[0] = flat
    wt_ref[0] = wts


def _three_nn(unknown, known, known_feats, W):
    return pl.pallas_call(
        _three_nn_body,
        grid=(B, ANB),
        in_specs=[
            pl.BlockSpec((1, ABLK, 3), lambda b, n: (b, n, 0)),
            pl.BlockSpec((1, M, 3), lambda b, n: (b, 0, 0)),
            pl.BlockSpec((1, C2, M), lambda b, n: (b, 0, 0)),
            pl.BlockSpec((CO, C1 + C2), lambda b, n: (0, 0)),
        ],
        out_specs=[
            pl.BlockSpec((1, 3, ABLK), lambda b, n: (b, 0, n)),
            pl.BlockSpec((1, 3, ABLK), lambda b, n: (b, 0, n)),
            pl.BlockSpec((1, M, CO), lambda b, n: (b, 0, 0)),
        ],
        out_shape=[
            jax.ShapeDtypeStruct((B, 3, N), jnp.int32),
            jax.ShapeDtypeStruct((B, 3, N), jnp.float32),
            jax.ShapeDtypeStruct((B, M, CO), jnp.float32),
        ],
    )(unknown, known, known_feats, W)


_NW = 32
_PTS_W = B * N // _NW
_NPART = N // _PTS_W
_CHUNK = 64
_NCH = _PTS_W // _CHUNK


def _sc_gather_body(pt_hbm, fi_hbm, wt_hbm, out_hbm, idx_s, w_s, rows_s,
                    out_v, sems):
    wid = lax.axis_index("s") * 2 + lax.axis_index("c")
    bb = wid // _NPART
    n0 = (wid % _NPART) * _PTS_W

    def issue(c, k):
        off = bb * 3 * N + n0 + c * _CHUNK
        for j in range(3):
            pltpu.sync_copy(fi_hbm.at[pl.ds(off + j * N, _CHUNK)],
                            idx_s[k][j])
            pltpu.async_copy(pt_hbm.at[idx_s[k][j]], rows_s[k][j],
                             sems[k][j])
            pltpu.sync_copy(wt_hbm.at[pl.ds(off + j * N, _CHUNK)],
                            w_s[k][j])

    def compute_write(c, k):
        for j in range(3):
            pltpu.make_async_copy(pt_hbm.at[idx_s[k][j]], rows_s[k][j],
                                  sems[k][j]).wait()
        r0, r1, r2 = rows_s[k]
        w0, w1, w2 = w_s[k]

        def grp(g, _):
            w0v = w0[pl.ds(g * 16, 16)]
            w1v = w1[pl.ds(g * 16, 16)]
            w2v = w2[pl.ds(g * 16, 16)]
            for p16 in range(16):
                p = g * 16 + p16
                for ct in range(CO // 16):
                    sl = pl.ds(ct * 16, 16)
                    out_v[p, sl] = (w0v[p16] * r0[p, sl]
                                    + w1v[p16] * r1[p, sl]
                                    + w2v[p16] * r2[p, sl])
            return 0

        lax.fori_loop(0, _CHUNK // 16, grp, 0)
        pltpu.sync_copy(out_v,
                        out_hbm.at[pl.ds(wid * _PTS_W + c * _CHUNK, _CHUNK)])

    issue(0, 0)

    def pair(i, _):
        c0 = i * 2
        issue(c0 + 1, 1)
        compute_write(c0, 0)

        @pl.when(c0 + 2 < _NCH)
        def _():
            issue(c0 + 2, 0)

        compute_write(c0 + 1, 1)
        return 0

    lax.fori_loop(0, _NCH // 2, pair, 0)


def _sc_gather(pt_flat, fi_flat, wt_flat):
    mesh = plsc.VectorSubcoreMesh(core_axis_name="c", subcore_axis_name="s")
    idx_t = [[pltpu.VMEM((_CHUNK,), jnp.int32) for _ in range(3)]
             for _ in range(2)]
    w_t = [[pltpu.VMEM((_CHUNK,), jnp.float32) for _ in range(3)]
           for _ in range(2)]
    rows_t = [[pltpu.VMEM((_CHUNK, CO), jnp.float32) for _ in range(3)]
              for _ in range(2)]
    f = pl.kernel(
        _sc_gather_body,
        out_type=jax.ShapeDtypeStruct((B * N, CO), jnp.float32),
        mesh=mesh,
        scratch_types=[
            idx_t,
            w_t,
            rows_t,
            pltpu.VMEM((_CHUNK, CO), jnp.float32),
            [[pltpu.SemaphoreType.DMA for _ in range(3)] for _ in range(2)],
        ],
    )
    return f(pt_flat, fi_flat, wt_flat)


def _mix_bn_body(g_ref, uf_ref, w_ref, gm_ref, bt_ref, out_ref,
                 h_scr, acc_scr):
    p = pl.program_id(0)
    b = pl.program_id(1)
    nb = pl.program_id(2)

    @pl.when(p == 0)
    def _():
        first = jnp.logical_and(b == 0, nb == 0)
        uf = uf_ref[0]
        w1 = w_ref[:, C2:]
        ht = lax.dot_general(
            w1, uf, (((1,), (0,)), ((), ())),
            preferred_element_type=jnp.float32,
        )
        ri = lax.broadcasted_iota(jnp.int32, (CO, CO), 0)
        ci = lax.broadcasted_iota(jnp.int32, (CO, CO), 1)
        ident = jnp.where(ri == ci, 1.0, 0.0).astype(jnp.float32)
        ht = ht + lax.dot_general(
            ident, g_ref[0], (((1,), (1,)), ((), ())),
            preferred_element_type=jnp.float32,
        )
        h_scr[pl.ds(b * CO, CO), pl.ds(nb * BLK, BLK)] = ht
        s = jnp.sum(ht, axis=1, keepdims=True)
        sq = jnp.sum(ht * ht, axis=1, keepdims=True)
        part = jnp.concatenate(
            [s, sq, jnp.zeros((CO, 6), jnp.float32)], axis=1)

        @pl.when(first)
        def _():
            acc_scr[...] = jnp.zeros_like(acc_scr)

        acc_scr[...] += part

    @pl.when(p == 1)
    def _():
        cnt = float(B * N)
        mean = acc_scr[:, 0:1] / cnt
        ex2 = acc_scr[:, 1:2] / cnt
        var = ex2 - mean * mean
        scale = gm_ref[...] * lax.rsqrt(var + 1e-5)
        shift = bt_ref[...] - mean * scale
        ht = h_scr[pl.ds(b * CO, CO), pl.ds(nb * BLK, BLK)]
        out_ref[0] = jnp.maximum(ht * scale + shift, 0.0)


def _mix_bn(g3, unknow_feats, W, gamma_c, beta_c):
    return pl.pallas_call(
        _mix_bn_body,
        grid=(2, B, NB),
        in_specs=[
            pl.BlockSpec((1, BLK, CO), lambda p, b, n: (b * (1 - p), n * (1 - p), 0)),
            pl.BlockSpec((1, C1, BLK), lambda p, b, n: (b * (1 - p), 0, n * (1 - p))),
            pl.BlockSpec((CO, C1 + C2), lambda p, b, n: (0, 0)),
            pl.BlockSpec((CO, 1), lambda p, b, n: (0, 0)),
            pl.BlockSpec((CO, 1), lambda p, b, n: (0, 0)),
        ],
        out_specs=pl.BlockSpec((1, CO, BLK), lambda p, b, n: (b * p, 0, n * p)),
        out_shape=jax.ShapeDtypeStruct((B, CO, N), jnp.float32),
        scratch_shapes=[
            pltpu.VMEM((B * CO, N), jnp.float32),
            pltpu.VMEM((CO, 8), jnp.float32),
        ],
    )(g3, unknow_feats, W, gamma_c, beta_c)


@jax.jit
def kernel(unknown, known, unknow_feats, known_feats, W, gamma, beta):
    fi, wts, pt = _three_nn(unknown, known, known_feats, W)
    g = _sc_gather(pt.reshape(B * M, CO), fi.reshape(B * 3 * N),
                   wts.reshape(B * 3 * N))
    g3 = g.reshape(B, N, CO)
    return _mix_bn(g3, unknow_feats, W,
                   gamma.reshape(CO, 1), beta.reshape(CO, 1))


# speedup vs baseline: 1.1113x; 1.1113x over previous
import functools

import jax
import jax.numpy as jnp
from jax import lax
from jax.experimental import pallas as pl
from jax.experimental.pallas import tpu as pltpu
from jax.experimental.pallas import tpu_sc as plsc

B, N, M = 8, 4096, 1024
C1, C2 = 128, 256
CO = 128
BLK = 1024
NB = N // BLK
ABLK = 1024
ANB = N // ABLK


def _three_nn_body(u_ref, kn_ref, kf_ref, w_ref, fi_ref, wt_ref, pt_ref):
    b = pl.program_id(0)
    nb = pl.program_id(1)

    @pl.when(nb == 0)
    def _():
        pt_ref[0] = lax.dot_general(
            kf_ref[0], w_ref[:, :C2], (((0,), (1,)), ((), ())),
            preferred_element_type=jnp.float32,
        )

    U = u_ref[0]
    K = kn_ref[0]
    kn2 = jnp.sum(K * K, axis=1, keepdims=True)
    G = lax.dot_general(
        K, U, (((1,), (1,)), ((), ())),
        preferred_element_type=jnp.float32,
        precision=lax.Precision.HIGHEST,
    )
    d2 = kn2 - 2.0 * G
    u2 = lax.dot_general(
        jnp.ones((1, 3), jnp.float32), U * U, (((1,), (1,)), ((), ())),
        preferred_element_type=jnp.float32,
        precision=lax.Precision.HIGHEST,
    )
    iota = lax.broadcasted_iota(jnp.int32, (M, ABLK), 0)
    masks, recips = [], []
    for _ in range(3):
        dmin = jnp.min(d2, axis=0, keepdims=True)
        eq = d2 == dmin
        d2 = jnp.where(eq, 1e30, d2)
        dist = jnp.sqrt(jnp.maximum(dmin + u2, 1e-12))
        masks.append(eq)
        recips.append(1.0 / (dist + 1e-8))
    packed = (
        jnp.where(masks[0], iota, 0)
        + jnp.where(masks[1], iota * 1024, 0)
        + jnp.where(masks[2], iota * (1024 * 1024), 0)
    )
    psum = jnp.sum(packed, axis=0, keepdims=True)
    idxs = [psum & 1023, (psum >> 10) & 1023, (psum >> 20) & 1023]
    norm = recips[0] + recips[1] + recips[2]
    flat = jnp.clip(jnp.concatenate(idxs, axis=0), 0, M - 1) + b * M
    wts = jnp.concatenate(recips, axis=0) / norm
    fi_ref[0] = flat
    wt_ref[0] = wts


def _three_nn(unknown, known, known_feats, W):
    return pl.pallas_call(
        _three_nn_body,
        grid=(B, ANB),
        in_specs=[
            pl.BlockSpec((1, ABLK, 3), lambda b, n: (b, n, 0)),
            pl.BlockSpec((1, M, 3), lambda b, n: (b, 0, 0)),
            pl.BlockSpec((1, C2, M), lambda b, n: (b, 0, 0)),
            pl.BlockSpec((CO, C1 + C2), lambda b, n: (0, 0)),
        ],
        out_specs=[
            pl.BlockSpec((1, 3, ABLK), lambda b, n: (b, 0, n)),
            pl.BlockSpec((1, 3, ABLK), lambda b, n: (b, 0, n)),
            pl.BlockSpec((1, M, CO), lambda b, n: (b, 0, 0)),
        ],
        out_shape=[
            jax.ShapeDtypeStruct((B, 3, N), jnp.int32),
            jax.ShapeDtypeStruct((B, 3, N), jnp.float32),
            jax.ShapeDtypeStruct((B, M, CO), jnp.float32),
        ],
    )(unknown, known, known_feats, W)


_TOTAL_ROWS = B * 3 * N
_NW = 32
_PER_W = _TOTAL_ROWS // _NW
_CHUNK = 384
_NCH = _PER_W // _CHUNK


def _sc_gather_body(pt_hbm, fi_hbm, out_hbm, idx0_v, idx1_v, rows0_v,
                    rows1_v, sem0, sem1):
    wid = lax.axis_index("s") * 2 + lax.axis_index("c")
    base = wid * _PER_W
    idxs = [idx0_v, idx1_v]
    rows = [rows0_v, rows1_v]
    sems = [sem0, sem1]

    def start_gather(c):
        k = c % 2
        pltpu.sync_copy(fi_hbm.at[pl.ds(base + c * _CHUNK, _CHUNK)],
                        idxs[k])
        return pltpu.async_copy(pt_hbm.at[idxs[k]], rows[k], sems[k])

    def drain(c, cp):
        cp.wait()
        pltpu.sync_copy(rows[c % 2],
                        out_hbm.at[pl.ds(base + c * _CHUNK, _CHUNK)])

    cps = [start_gather(0)]
    for c in range(1, _NCH):
        cps.append(start_gather(c))
        drain(c - 1, cps[c - 1])
    drain(_NCH - 1, cps[_NCH - 1])


def _sc_gather(pt_flat, fi_flat):
    mesh = plsc.VectorSubcoreMesh(core_axis_name="c", subcore_axis_name="s")
    f = pl.kernel(
        _sc_gather_body,
        out_type=jax.ShapeDtypeStruct((_TOTAL_ROWS, CO), jnp.float32),
        mesh=mesh,
        scratch_types=[
            pltpu.VMEM((_CHUNK,), jnp.int32),
            pltpu.VMEM((_CHUNK,), jnp.int32),
            pltpu.VMEM((_CHUNK, CO), jnp.float32),
            pltpu.VMEM((_CHUNK, CO), jnp.float32),
            pltpu.SemaphoreType.DMA,
            pltpu.SemaphoreType.DMA,
        ],
    )
    return f(pt_flat, fi_flat)


def _mix_bn_body(g_ref, wt_ref, uf_ref, w_ref, gm_ref, bt_ref, out_ref,
                 h_scr, acc_scr):
    p = pl.program_id(0)
    b = pl.program_id(1)
    nb = pl.program_id(2)

    @pl.when(p == 0)
    def _():
        first = jnp.logical_and(b == 0, nb == 0)
        uf = uf_ref[0]
        w1 = w_ref[:, C2:]
        wts = wt_ref[0]
        ht = lax.dot_general(
            w1, uf, (((1,), (0,)), ((), ())),
            preferred_element_type=jnp.float32,
        )
        ri = lax.broadcasted_iota(jnp.int32, (CO, CO), 0)
        ci = lax.broadcasted_iota(jnp.int32, (CO, CO), 1)
        ident = jnp.where(ri == ci, 1.0, 0.0).astype(jnp.float32)
        for j in range(3):
            gj = g_ref[0, j]
            gjt = lax.dot_general(
                ident, gj, (((1,), (1,)), ((), ())),
                preferred_element_type=jnp.float32,
            )
            ht = ht + gjt * wts[j : j + 1, :]
        h_scr[pl.ds(b * CO, CO), pl.ds(nb * BLK, BLK)] = ht
        s = jnp.sum(ht, axis=1, keepdims=True)
        sq = jnp.sum(ht * ht, axis=1, keepdims=True)
        part = jnp.concatenate(
            [s, sq, jnp.zeros((CO, 6), jnp.float32)], axis=1)

        @pl.when(first)
        def _():
            acc_scr[...] = jnp.zeros_like(acc_scr)

        acc_scr[...] += part

    @pl.when(p == 1)
    def _():
        cnt = float(B * N)
        mean = acc_scr[:, 0:1] / cnt
        ex2 = acc_scr[:, 1:2] / cnt
        var = ex2 - mean * mean
        scale = gm_ref[...] * lax.rsqrt(var + 1e-5)
        shift = bt_ref[...] - mean * scale
        ht = h_scr[pl.ds(b * CO, CO), pl.ds(nb * BLK, BLK)]
        out_ref[0] = jnp.maximum(ht * scale + shift, 0.0)


def _mix_bn(g4, wts, unknow_feats, W, gamma_c, beta_c):
    return pl.pallas_call(
        _mix_bn_body,
        grid=(2, B, NB),
        in_specs=[
            pl.BlockSpec((1, 3, BLK, CO), lambda p, b, n: (b * (1 - p), 0, n * (1 - p), 0)),
            pl.BlockSpec((1, 3, BLK), lambda p, b, n: (b * (1 - p), 0, n * (1 - p))),
            pl.BlockSpec((1, C1, BLK), lambda p, b, n: (b * (1 - p), 0, n * (1 - p))),
            pl.BlockSpec((CO, C1 + C2), lambda p, b, n: (0, 0)),
            pl.BlockSpec((CO, 1), lambda p, b, n: (0, 0)),
            pl.BlockSpec((CO, 1), lambda p, b, n: (0, 0)),
        ],
        out_specs=pl.BlockSpec((1, CO, BLK), lambda p, b, n: (b * p, 0, n * p)),
        out_shape=jax.ShapeDtypeStruct((B, CO, N), jnp.float32),
        scratch_shapes=[
            pltpu.VMEM((B * CO, N), jnp.float32),
            pltpu.VMEM((CO, 8), jnp.float32),
        ],
    )(g4, wts, unknow_feats, W, gamma_c, beta_c)


@jax.jit
def kernel(unknown, known, unknow_feats, known_feats, W, gamma, beta):
    fi, wts, pt = _three_nn(unknown, known, known_feats, W)
    g = _sc_gather(pt.reshape(B * M, CO), fi.reshape(_TOTAL_ROWS))
    g4 = g.reshape(B, 3, N, CO)
    return _mix_bn(g4, wts, unknow_feats, W,
                   gamma.reshape(CO, 1), beta.reshape(CO, 1))


# speedup vs baseline: 1.2115x; 1.0902x over previous
import jax
import jax.numpy as jnp
from jax import lax
from jax.experimental import pallas as pl
from jax.experimental.pallas import tpu as pltpu
from jax.experimental.pallas import tpu_sc as plsc

B, N, M = 8, 4096, 1024
C1, C2 = 128, 256
CO = 128
BLK = 1024
NB = N // BLK
ABLK = 1024
ANB = N // ABLK


def _three_nn_body(u_ref, kn_ref, kf_ref, w_ref, fi_ref, wt_ref, pt_ref):
    b = pl.program_id(0)
    nb = pl.program_id(1)

    @pl.when(nb == 0)
    def _():
        pt_ref[0] = lax.dot_general(
            kf_ref[0], w_ref[:, :C2], (((0,), (1,)), ((), ())),
            preferred_element_type=jnp.float32,
        )

    U = u_ref[0]
    K = kn_ref[0]
    i3r = lax.broadcasted_iota(jnp.int32, (3, 3), 0)
    i3c = lax.broadcasted_iota(jnp.int32, (3, 3), 1)
    ident3 = jnp.where(i3r == i3c, 1.0, 0.0).astype(jnp.float32)
    Ut = lax.dot_general(
        ident3, U, (((1,), (1,)), ((), ())),
        preferred_element_type=jnp.float32,
        precision=lax.Precision.HIGHEST,
    )
    d2 = None
    for c in range(3):
        diff = K[:, c : c + 1] - Ut[c : c + 1, :]
        sq = diff * diff
        d2 = sq if d2 is None else d2 + sq
    iota = lax.broadcasted_iota(jnp.int32, (M, ABLK), 0)
    masks, recips = [], []
    for _ in range(3):
        dmin = jnp.min(d2, axis=0, keepdims=True)
        eq = d2 == dmin
        d2 = jnp.where(eq, 1e30, d2)
        dist = jnp.sqrt(jnp.maximum(dmin, 1e-12))
        masks.append(eq)
        recips.append(1.0 / (dist + 1e-8))
    packed = (
        jnp.where(masks[0], iota, 0)
        + jnp.where(masks[1], iota * 1024, 0)
        + jnp.where(masks[2], iota * (1024 * 1024), 0)
    )
    psum = jnp.sum(packed, axis=0, keepdims=True)
    idxs = [psum & 1023, (psum >> 10) & 1023, (psum >> 20) & 1023]
    norm = recips[0] + recips[1] + recips[2]
    flat = jnp.clip(jnp.concatenate(idxs, axis=0), 0, M - 1) + b * M
    wts = jnp.concatenate(recips, axis=0) / norm
    fi_ref[0] = flat
    wt_ref[0] = wts


def _three_nn(unknown, known, known_feats, W):
    return pl.pallas_call(
        _three_nn_body,
        grid=(B, ANB),
        in_specs=[
            pl.BlockSpec((1, ABLK, 3), lambda b, n: (b, n, 0)),
            pl.BlockSpec((1, M, 3), lambda b, n: (b, 0, 0)),
            pl.BlockSpec((1, C2, M), lambda b, n: (b, 0, 0)),
            pl.BlockSpec((CO, C1 + C2), lambda b, n: (0, 0)),
        ],
        out_specs=[
            pl.BlockSpec((1, 3, ABLK), lambda b, n: (b, 0, n)),
            pl.BlockSpec((1, 3, ABLK), lambda b, n: (b, 0, n)),
            pl.BlockSpec((1, M, CO), lambda b, n: (b, 0, 0)),
        ],
        out_shape=[
            jax.ShapeDtypeStruct((B, 3, N), jnp.int32),
            jax.ShapeDtypeStruct((B, 3, N), jnp.float32),
            jax.ShapeDtypeStruct((B, M, CO), jnp.float32),
        ],
    )(unknown, known, known_feats, W)


_TOTAL_ROWS = B * 3 * N
_NW = 32
_PER_W = _TOTAL_ROWS // _NW
_CHUNK = 384
_NCH = _PER_W // _CHUNK


def _sc_gather_body(pt_hbm, fi_hbm, out_hbm, idx0_v, idx1_v, rows0_v,
                    rows1_v, sem0, sem1):
    wid = lax.axis_index("s") * 2 + lax.axis_index("c")
    base = wid * _PER_W
    idxs = [idx0_v, idx1_v]
    rows = [rows0_v, rows1_v]
    sems = [sem0, sem1]

    def start_gather(c):
        k = c % 2
        pltpu.sync_copy(fi_hbm.at[pl.ds(base + c * _CHUNK, _CHUNK)],
                        idxs[k])
        return pltpu.async_copy(pt_hbm.at[idxs[k]], rows[k], sems[k])

    def drain(c, cp):
        cp.wait()
        pltpu.sync_copy(rows[c % 2],
                        out_hbm.at[pl.ds(base + c * _CHUNK, _CHUNK)])

    cps = [start_gather(0)]
    for c in range(1, _NCH):
        cps.append(start_gather(c))
        drain(c - 1, cps[c - 1])
    drain(_NCH - 1, cps[_NCH - 1])


def _sc_gather(pt_flat, fi_flat):
    mesh = plsc.VectorSubcoreMesh(core_axis_name="c", subcore_axis_name="s")
    f = pl.kernel(
        _sc_gather_body,
        out_type=jax.ShapeDtypeStruct((_TOTAL_ROWS, CO), jnp.float32),
        mesh=mesh,
        scratch_types=[
            pltpu.VMEM((_CHUNK,), jnp.int32),
            pltpu.VMEM((_CHUNK,), jnp.int32),
            pltpu.VMEM((_CHUNK, CO), jnp.float32),
            pltpu.VMEM((_CHUNK, CO), jnp.float32),
            pltpu.SemaphoreType.DMA,
            pltpu.SemaphoreType.DMA,
        ],
    )
    return f(pt_flat, fi_flat)


def _mix_bn_body(g_ref, wt_ref, uf_ref, w_ref, gm_ref, bt_ref, out_ref,
                 h_scr, acc_scr):
    p = pl.program_id(0)
    b = pl.program_id(1)
    nb = pl.program_id(2)

    @pl.when(p == 0)
    def _():
        first = jnp.logical_and(b == 0, nb == 0)
        uf = uf_ref[0]
        w1 = w_ref[:, C2:]
        wts = wt_ref[0]
        ht = lax.dot_general(
            w1, uf, (((1,), (0,)), ((), ())),
            preferred_element_type=jnp.float32,
        )
        ri = lax.broadcasted_iota(jnp.int32, (CO, CO), 0)
        ci = lax.broadcasted_iota(jnp.int32, (CO, CO), 1)
        ident = jnp.where(ri == ci, 1.0, 0.0).astype(jnp.float32)
        for j in range(3):
            gj = g_ref[0, j]
            gjt = lax.dot_general(
                ident, gj, (((1,), (1,)), ((), ())),
                preferred_element_type=jnp.float32,
            )
            ht = ht + gjt * wts[j : j + 1, :]
        h_scr[pl.ds(b * CO, CO), pl.ds(nb * BLK, BLK)] = ht
        s = jnp.sum(ht, axis=1, keepdims=True)
        sq = jnp.sum(ht * ht, axis=1, keepdims=True)
        part = jnp.concatenate(
            [s, sq, jnp.zeros((CO, 6), jnp.float32)], axis=1)

        @pl.when(first)
        def _():
            acc_scr[...] = jnp.zeros_like(acc_scr)

        acc_scr[...] += part

    @pl.when(p == 1)
    def _():
        cnt = float(B * N)
        mean = acc_scr[:, 0:1] / cnt
        ex2 = acc_scr[:, 1:2] / cnt
        var = ex2 - mean * mean
        scale = gm_ref[...] * lax.rsqrt(var + 1e-5)
        shift = bt_ref[...] - mean * scale
        ht = h_scr[pl.ds(b * CO, CO), pl.ds(nb * BLK, BLK)]
        out_ref[0] = jnp.maximum(ht * scale + shift, 0.0)


def _mix_bn(g4, wts, unknow_feats, W, gamma_c, beta_c):
    return pl.pallas_call(
        _mix_bn_body,
        grid=(2, B, NB),
        in_specs=[
            pl.BlockSpec((1, 3, BLK, CO), lambda p, b, n: (b * (1 - p), 0, n * (1 - p), 0)),
            pl.BlockSpec((1, 3, BLK), lambda p, b, n: (b * (1 - p), 0, n * (1 - p))),
            pl.BlockSpec((1, C1, BLK), lambda p, b, n: (b * (1 - p), 0, n * (1 - p))),
            pl.BlockSpec((CO, C1 + C2), lambda p, b, n: (0, 0)),
            pl.BlockSpec((CO, 1), lambda p, b, n: (0, 0)),
            pl.BlockSpec((CO, 1), lambda p, b, n: (0, 0)),
        ],
        out_specs=pl.BlockSpec((1, CO, BLK), lambda p, b, n: (b * p, 0, n * p)),
        out_shape=jax.ShapeDtypeStruct((B, CO, N), jnp.float32),
        scratch_shapes=[
            pltpu.VMEM((B * CO, N), jnp.float32),
            pltpu.VMEM((CO, 8), jnp.float32),
        ],
    )(g4, wts, unknow_feats, W, gamma_c, beta_c)


@jax.jit
def kernel(unknown, known, unknow_feats, known_feats, W, gamma, beta):
    fi, wts, pt = _three_nn(unknown, known, known_feats, W)
    g = _sc_gather(pt.reshape(B * M, CO), fi.reshape(_TOTAL_ROWS))
    g4 = g.reshape(B, 3, N, CO)
    return _mix_bn(g4, wts, unknow_feats, W,
                   gamma.reshape(CO, 1), beta.reshape(CO, 1))
